# Initial kernel scaffold; baseline (speedup 1.0000x reference)
import jax
import jax.numpy as jnp
from jax.experimental import pallas as pl


def kernel(spatial_ht_list, temporal_ht_list, ts_mask, same_scene_mask, W1_w, W1_b, W2_w, W2_b):
    raise NotImplementedError("write your pallas kernel here")


# speedup vs baseline: 1.7022x; 1.7022x over previous
import functools

import jax
import jax.numpy as jnp
from jax.experimental import pallas as pl

N = 256
H = 256
A = 64
BI = 16


def _attn_block(s_ref, t_ref, ts_ref, ss_ref, w1_ref, b1_ref, w2_ref, b2_ref,
                out_ref):
    i = pl.program_id(0)
    ts = ts_ref[0, :]
    ss = ss_ref[0, :]
    m = jnp.logical_and(ts == 1.0, ss == 0.0).astype(jnp.float32)
    en = jnp.sum(m)

    tp2 = jax.lax.dot_general(
        t_ref[...], w2_ref[...], (((1,), (1,)), ((), ())),
        preferred_element_type=jnp.float32) + b2_ref[0, :][None, :]
    v = jax.lax.dot_general(
        tp2, w1_ref[...], (((1,), (0,)), ((), ())),
        preferred_element_type=jnp.float32)
    c = jnp.sum(tp2 * b1_ref[0, :][None, :], axis=1)

    s = s_ref[0]
    sm = jnp.sum(s * v[:, None, :], axis=2)
    scale = en * jax.lax.rsqrt(jnp.float32(A))
    logits = (sm + c[:, None]) * scale

    row_ids = i * BI + jax.lax.broadcasted_iota(jnp.int32, (BI, N), 0)
    col_ids = jax.lax.broadcasted_iota(jnp.int32, (BI, N), 1)
    off_diag = (row_ids != col_ids).astype(jnp.float32)
    m_rows = jnp.sum(jnp.where(row_ids == col_ids, m[None, :], 0.0), axis=1)
    num = jnp.exp(logits) * off_diag * m[None, :] * m_rows[:, None]
    den = jnp.sum(num, axis=1, keepdims=True)
    safe_den = jnp.where(den == 0.0, 1.0, den)
    score = num / safe_den

    out_ref[...] = jnp.sum(s * score[:, :, None], axis=1)


@jax.jit
def _edge_to_node_attention(spatial_ht_list, temporal_ht_list, ts_mask,
                            same_scene_mask, W1_w, W1_b, W2_w, W2_b):
    grid = (N // BI,)
    return pl.pallas_call(
        _attn_block,
        grid=grid,
        in_specs=[
            pl.BlockSpec((1, BI, N, H), lambda i: (0, i, 0, 0)),
            pl.BlockSpec((BI, H), lambda i: (i, 0)),
            pl.BlockSpec((1, N), lambda i: (0, 0)),
            pl.BlockSpec((1, N), lambda i: (0, 0)),
            pl.BlockSpec((A, H), lambda i: (0, 0)),
            pl.BlockSpec((1, A), lambda i: (0, 0)),
            pl.BlockSpec((A, H), lambda i: (0, 0)),
            pl.BlockSpec((1, A), lambda i: (0, 0)),
        ],
        out_specs=pl.BlockSpec((BI, H), lambda i: (i, 0)),
        out_shape=jax.ShapeDtypeStruct((N, H), jnp.float32),
    )(spatial_ht_list, temporal_ht_list, ts_mask, same_scene_mask,
      W1_w, W1_b, W2_w, W2_b)


def kernel(spatial_ht_list, temporal_ht_list, ts_mask, same_scene_mask,
           W1_w, W1_b, W2_w, W2_b):
    return _edge_to_node_attention(
        spatial_ht_list, temporal_ht_list,
        ts_mask.reshape(1, N), same_scene_mask.reshape(1, N),
        W1_w, W1_b.reshape(1, A), W2_w, W2_b.reshape(1, A))
